# Initial kernel scaffold; baseline (speedup 1.0000x reference)
import jax
import jax.numpy as jnp
from jax.experimental import pallas as pl


def kernel(x, edge_index, w1, b1, g1, be1, w2, b2, g2, be2, w3, b3, g3, be3, gcn_w, gcn_b, lin1_w, lin1_b, gate_w, gate_b, g4, be4):
    raise NotImplementedError("write your pallas kernel here")


# speedup vs baseline: 5.1870x; 5.1870x over previous
import functools

import jax
import jax.numpy as jnp
from jax import lax
from jax.experimental import pallas as pl
from jax.experimental.pallas import tpu as pltpu
from jax.experimental.pallas import tpu_sc as plsc

_N = 10000
_C = 4
_L = 1201
_E = 640000
_REP = 48
_HID = 128
_T2 = 298
_T3 = 48
_EPS = 1e-5

_NPAD = 10240
_NW = 32
_CHUNK = 128
_NCH = 160
_EW = _NCH * _CHUNK
_EPADDED = _NW * _EW
_RPT = _NPAD // 16

_BN1 = 16
_BN2 = 40
_BN3 = 40
_BVP = 1280
_BVF = 2000


def _moments_body(x_ref, out_ref):
    @pl.when(pl.program_id(0) == 0)
    def _init():
        out_ref[...] = jnp.zeros_like(out_ref)

    v = x_ref[...]
    out_ref[0:4, :] += jnp.sum(v, axis=0)
    for c in range(4):
        out_ref[4 + 4 * c:8 + 4 * c, :] += jnp.sum(v * v[:, c:c + 1, :], axis=0)


def _moments(x):
    return pl.pallas_call(
        _moments_body,
        grid=(_N // _BN1,),
        in_specs=[pl.BlockSpec((_BN1, _C, _L), lambda i: (i, 0, 0))],
        out_specs=pl.BlockSpec((20, _L), lambda i: (0, 0)),
        out_shape=jax.ShapeDtypeStruct((20, _L), jnp.float32),
    )(x)


def _conv2_body(x_ref, w_ref, b_ref, h2_ref, st_ref):
    @pl.when(pl.program_id(0) == 0)
    def _init():
        st_ref[...] = jnp.zeros_like(st_ref)

    accs = [None] * 4
    for c in range(4):
        for k in range(11):
            a, r = k // 4, k % 4
            xk = x_ref[:, 4 * r + c, a:a + _T2]
            for o in range(4):
                t = w_ref[o, c, k] * xk
                accs[o] = t if accs[o] is None else accs[o] + t
    for o in range(4):
        h = accs[o] + b_ref[o]
        h2_ref[:, o, 0:_T2] = h
        h2_ref[:, o, _T2:300] = jnp.zeros((_BN2, 2), jnp.float32)
        st_ref[o, :] += jnp.sum(h, axis=0)
        st_ref[4 + o, :] += jnp.sum(h * h, axis=0)


def _conv2(xs, w2f, b2f):
    return pl.pallas_call(
        _conv2_body,
        grid=(_N // _BN2,),
        in_specs=[
            pl.BlockSpec((_BN2, 16, 301), lambda i: (i, 0, 0)),
            pl.BlockSpec(memory_space=pltpu.SMEM),
            pl.BlockSpec(memory_space=pltpu.SMEM),
        ],
        out_specs=[
            pl.BlockSpec((_BN2, _C, 300), lambda i: (i, 0, 0)),
            pl.BlockSpec((8, _T2), lambda i: (0, 0)),
        ],
        out_shape=[
            jax.ShapeDtypeStruct((_N, _C, 300), jnp.float32),
            jax.ShapeDtypeStruct((8, _T2), jnp.float32),
        ],
    )(xs, w2f, b2f)


def _conv3_body(h2_ref, w3b_ref, sc_ref, hp_ref, st_ref):
    @pl.when(pl.program_id(0) == 0)
    def _init():
        st_ref[...] = jnp.zeros_like(st_ref)

    acc = None
    for c in range(4):
        rc = jnp.maximum(sc_ref[c] * h2_ref[:, c, :] + sc_ref[4 + c], 0.0)
        t = jnp.dot(rc[:, 0:_T2], w3b_ref[c],
                    preferred_element_type=jnp.float32)
        acc = t if acc is None else acc + t
    h3 = acc + sc_ref[8]
    hp_ref[...] = h3
    st_ref[0, :] += jnp.sum(h3, axis=0)
    st_ref[1, :] += jnp.sum(h3 * h3, axis=0)


def _conv3(h2, w3band, scal):
    return pl.pallas_call(
        _conv3_body,
        grid=(_N // _BN3,),
        in_specs=[
            pl.BlockSpec((_BN3, _C, 300), lambda i: (i, 0, 0)),
            pl.BlockSpec((_C, _T2, _T3), lambda i: (0, 0, 0)),
            pl.BlockSpec(memory_space=pltpu.SMEM),
        ],
        out_specs=[
            pl.BlockSpec((_BN3, _T3), lambda i: (i, 0)),
            pl.BlockSpec((2, _T3), lambda i: (0, 0)),
        ],
        out_shape=[
            jax.ShapeDtypeStruct((_N, _T3), jnp.float32),
            jax.ShapeDtypeStruct((2, _T3), jnp.float32),
        ],
    )(h2, w3band, scal)


def _deg_sc(dstp, z16, ones16):
    mesh = plsc.VectorSubcoreMesh(core_axis_name="c", subcore_axis_name="s")

    @functools.partial(
        pl.kernel,
        out_type=jax.ShapeDtypeStruct((2, _NPAD, 16), jnp.float32),
        mesh=mesh,
        compiler_params=pltpu.CompilerParams(use_tc_tiling_on_sc=False),
        scratch_types=[
            pltpu.VMEM((_NCH, _CHUNK), jnp.int32),
            pltpu.VMEM((_CHUNK, 16), jnp.float32),
            pltpu.VMEM_SHARED((_NPAD, 16), jnp.float32),
        ],
    )
    def k(dst_hbm, z_hbm, ones_hbm, out_hbm, dstv, onesv, accsh):
        ci = lax.axis_index("c")
        si = lax.axis_index("s")
        w = si * 2 + ci
        r0 = si * _RPT
        pltpu.sync_copy(z_hbm.at[pl.ds(r0, _RPT)], accsh.at[pl.ds(r0, _RPT)])
        pltpu.sync_copy(ones_hbm, onesv)
        pltpu.sync_copy(dst_hbm.at[w], dstv)
        plsc.subcore_barrier()

        def body(j):
            pltpu.sync_copy(onesv, accsh.at[dstv.at[j]], add=True)

        pl.loop(0, _NCH)(body)
        plsc.subcore_barrier()
        pltpu.sync_copy(accsh.at[pl.ds(r0, _RPT)],
                        out_hbm.at[ci, pl.ds(r0, _RPT)])

    return k(dstp, z16, ones16)


def _prop_sc(srcp, dstp, hp, z48):
    mesh = plsc.VectorSubcoreMesh(core_axis_name="c", subcore_axis_name="s")

    @functools.partial(
        pl.kernel,
        out_type=jax.ShapeDtypeStruct((2, _NPAD, _REP), jnp.float32),
        mesh=mesh,
        compiler_params=pltpu.CompilerParams(use_tc_tiling_on_sc=False),
        scratch_types=[
            pltpu.VMEM((_NCH, _CHUNK), jnp.int32),
            pltpu.VMEM((_NCH, _CHUNK), jnp.int32),
            pltpu.VMEM((_CHUNK, _REP), jnp.float32),
            pltpu.VMEM((_CHUNK, _REP), jnp.float32),
            pltpu.VMEM_SHARED((_NPAD, _REP), jnp.float32),
            pltpu.SemaphoreType.DMA,
            pltpu.SemaphoreType.DMA,
        ],
    )
    def k(src_hbm, dst_hbm, hp_hbm, z_hbm, out_hbm,
          srcv, dstv, rows0, rows1, accsh, sem0, sem1):
        ci = lax.axis_index("c")
        si = lax.axis_index("s")
        w = si * 2 + ci
        r0 = si * _RPT
        pltpu.sync_copy(z_hbm.at[pl.ds(r0, _RPT)], accsh.at[pl.ds(r0, _RPT)])
        pltpu.sync_copy(src_hbm.at[w], srcv)
        pltpu.sync_copy(dst_hbm.at[w], dstv)
        plsc.subcore_barrier()

        bufs = (rows0, rows1)
        sems = (sem0, sem1)

        def body(j):
            cps = [None, None]
            for b in range(2):
                cps[b] = pltpu.async_copy(
                    hp_hbm.at[srcv.at[j + b]], bufs[b], sems[b])
            for b in range(2):
                cps[b].wait()
                pltpu.sync_copy(bufs[b], accsh.at[dstv.at[j + b]], add=True)

        pl.loop(0, _NCH, step=2)(body)
        plsc.subcore_barrier()
        pltpu.sync_copy(accsh.at[pl.ds(r0, _RPT)],
                        out_hbm.at[ci, pl.ds(r0, _RPT)])

    return k(srcp, dstp, hp, z48)


def _prep_body(degp_ref, hpre_ref, sc_ref, h_ref, hp_ref, dinv_ref):
    d = degp_ref[0] + degp_ref[1] + 1.0
    dinv = lax.rsqrt(d)
    dcol = dinv[:, 0:1]
    h = sc_ref[0] * hpre_ref[...] + sc_ref[1]
    h_ref[...] = h
    hp_ref[...] = h * dcol
    dinv_ref[...] = dcol


def _prep(degp, hpre_pad, scal):
    return pl.pallas_call(
        _prep_body,
        grid=(_NPAD // _BVP,),
        in_specs=[
            pl.BlockSpec((2, _BVP, 16), lambda i: (0, i, 0)),
            pl.BlockSpec((_BVP, _REP), lambda i: (i, 0)),
            pl.BlockSpec(memory_space=pltpu.SMEM),
        ],
        out_specs=[
            pl.BlockSpec((_BVP, _REP), lambda i: (i, 0)),
            pl.BlockSpec((_BVP, _REP), lambda i: (i, 0)),
            pl.BlockSpec((_BVP, 1), lambda i: (i, 0)),
        ],
        out_shape=[
            jax.ShapeDtypeStruct((_NPAD, _REP), jnp.float32),
            jax.ShapeDtypeStruct((_NPAD, _REP), jnp.float32),
            jax.ShapeDtypeStruct((_NPAD, 1), jnp.float32),
        ],
    )(degp, hpre_pad, scal)


def _fuse_body(h_ref, acc_ref, dinv_ref, gw_ref, gb_ref, lw_ref, lb_ref,
               qw_ref, qb_ref, o_ref, st_ref):
    @pl.when(pl.program_id(0) == 0)
    def _init():
        st_ref[...] = jnp.zeros_like(st_ref)

    dinv = dinv_ref[...]
    h = h_ref[...]
    a = (acc_ref[0] + acc_ref[1]) * dinv + (dinv * dinv) * h
    z = jnp.tanh(jnp.dot(a, gw_ref[...], preferred_element_type=jnp.float32)
                 + gb_ref[...])
    g = jax.nn.sigmoid(
        jnp.dot(z, qw_ref[...], preferred_element_type=jnp.float32)
        + qb_ref[...])
    xl = jnp.dot(h, lw_ref[...], preferred_element_type=jnp.float32) \
        + lb_ref[...]
    o = jnp.maximum((1.0 - g) * xl + g * z, 0.0)
    o_ref[...] = o
    st_ref[0, :] += jnp.sum(o, axis=0)
    st_ref[1, :] += jnp.sum(o * o, axis=0)


def _fuse(h, accp, dinv, gcn_w, gcn_b, lin1_w, lin1_b, gate_w, gate_b):
    return pl.pallas_call(
        _fuse_body,
        grid=(_N // _BVF,),
        in_specs=[
            pl.BlockSpec((_BVF, _REP), lambda i: (i, 0)),
            pl.BlockSpec((2, _BVF, _REP), lambda i: (0, i, 0)),
            pl.BlockSpec((_BVF, 1), lambda i: (i, 0)),
            pl.BlockSpec((_REP, _HID), lambda i: (0, 0)),
            pl.BlockSpec((1, _HID), lambda i: (0, 0)),
            pl.BlockSpec((_REP, _HID), lambda i: (0, 0)),
            pl.BlockSpec((1, _HID), lambda i: (0, 0)),
            pl.BlockSpec((_HID, _HID), lambda i: (0, 0)),
            pl.BlockSpec((1, _HID), lambda i: (0, 0)),
        ],
        out_specs=[
            pl.BlockSpec((_BVF, _HID), lambda i: (i, 0)),
            pl.BlockSpec((2, _HID), lambda i: (0, 0)),
        ],
        out_shape=[
            jax.ShapeDtypeStruct((_N, _HID), jnp.float32),
            jax.ShapeDtypeStruct((2, _HID), jnp.float32),
        ],
    )(h, accp, dinv, gcn_w, gcn_b, lin1_w, lin1_b, gate_w, gate_b)


def _bn4_body(o_ref, ac_ref, out_ref):
    out_ref[...] = ac_ref[0:1, :] * o_ref[...] + ac_ref[1:2, :]


def _bn4(o, a4c4):
    return pl.pallas_call(
        _bn4_body,
        grid=(_N // _BVF,),
        in_specs=[
            pl.BlockSpec((_BVF, _HID), lambda i: (i, 0)),
            pl.BlockSpec((2, _HID), lambda i: (0, 0)),
        ],
        out_specs=pl.BlockSpec((_BVF, _HID), lambda i: (i, 0)),
        out_shape=jax.ShapeDtypeStruct((_N, _HID), jnp.float32),
    )(o, a4c4)


def kernel(x, edge_index, w1, b1, g1, be1, w2, b2, g2, be2, w3, b3, g3, be3,
           gcn_w, gcn_b, lin1_w, lin1_b, gate_w, gate_b, g4, be4):
    f32 = jnp.float32
    x = x.astype(f32)

    src = edge_index[0].astype(jnp.int32)
    dst = edge_index[1].astype(jnp.int32)
    npadidx = (_N + jnp.arange(_EPADDED - _E, dtype=jnp.int32)
               % (_NPAD - _N)).astype(jnp.int32)
    srcp = jnp.concatenate([src, npadidx]).reshape(_NW, _NCH, _CHUNK)
    dstp = jnp.concatenate([dst, npadidx]).reshape(_NW, _NCH, _CHUNK)
    z16 = jnp.zeros((_NPAD, 16), f32)
    z48 = jnp.zeros((_NPAD, _REP), f32)
    ones16 = jnp.ones((_CHUNK, 16), f32)

    degp = _deg_sc(dstp, z16, ones16)

    mom = _moments(x)
    cnt1 = float(_N * _L)
    s1 = jnp.sum(mom[0:4, :], axis=1)
    s2 = jnp.sum(mom[4:20, :], axis=1).reshape(4, 4)
    mu = s1 / cnt1
    cov = s2 / cnt1 - mu[:, None] * mu[None, :]
    w1m = w1[:, :, 0]
    mean1 = w1m @ mu + b1
    var1 = jnp.sum((w1m @ cov) * w1m, axis=1)
    a1 = g1 * lax.rsqrt(var1 + _EPS)
    c1 = be1 - a1 * mean1
    a1w = a1[:, None] * w1m
    d1 = a1 * b1 + c1
    w2f = jnp.einsum('oik,ic->ock', w2, a1w)
    b2f = b2 + jnp.einsum('oik,i->o', w2, d1)

    x_pad = jnp.pad(x, ((0, 0), (0, 0), (0, 3)))
    xs = x_pad.reshape(_N, _C, 301, 4).transpose(0, 3, 1, 2) \
        .reshape(_N, 16, 301)
    h2, st2 = _conv2(xs, w2f, b2f)
    cnt2 = float(_N * _T2)
    m2 = jnp.sum(st2[0:4, :], axis=1) / cnt2
    v2 = jnp.sum(st2[4:8, :], axis=1) / cnt2 - m2 * m2
    a2 = g2 * lax.rsqrt(v2 + _EPS)
    c2 = be2 - a2 * m2

    kk = jnp.arange(_T2)[:, None] - 6 * jnp.arange(_T3)[None, :]
    band = (kk >= 0) & (kk < 11)
    w3band = jnp.where(band[None, :, :],
                       w3[0][:, jnp.clip(kk, 0, 10)], 0.0).astype(f32)
    scal3 = jnp.concatenate([a2, c2, b3]).astype(f32)
    hpre, st3 = _conv3(h2, w3band, scal3)
    cnt3 = float(_N * _T3)
    m3 = jnp.sum(st3[0, :]) / cnt3
    v3 = jnp.sum(st3[1, :]) / cnt3 - m3 * m3
    a3 = g3[0] * lax.rsqrt(v3 + _EPS)
    c3 = be3[0] - a3 * m3

    hpre_pad = jnp.pad(hpre, ((0, _NPAD - _N), (0, 0)))
    scal_p = jnp.stack([a3, c3]).astype(f32)
    h, hp, dinv = _prep(degp, hpre_pad, scal_p)

    accp = _prop_sc(srcp, dstp, hp, z48)

    o, st4 = _fuse(h, accp, dinv, gcn_w, gcn_b.reshape(1, _HID),
                   lin1_w, lin1_b.reshape(1, _HID),
                   gate_w, gate_b.reshape(1, _HID))
    m4 = st4[0, :] / float(_N)
    v4 = st4[1, :] / float(_N) - m4 * m4
    a4 = g4 * lax.rsqrt(v4 + _EPS)
    c4 = be4 - a4 * m4
    a4c4 = jnp.stack([a4, c4])

    return _bn4(o, a4c4)
